# baseline (speedup 1.0000x reference)
import functools

import jax
import jax.numpy as jnp
from jax import lax
from jax.experimental import pallas as pl
from jax.experimental.pallas import tpu as pltpu
from jax.experimental.pallas import tpu_sc as plsc

EMBED_DIM = 64
CHUNK = 128
NBUF = 8
NUM_CORES = 2
NUM_SUBCORES = 16
NUM_WORKERS = NUM_CORES * NUM_SUBCORES


@functools.lru_cache(maxsize=None)
def _build_gather(total: int):
    nchunk_total = total // CHUNK
    nchunk = nchunk_total // NUM_WORKERS
    mesh = plsc.VectorSubcoreMesh(core_axis_name="c", subcore_axis_name="s")

    nchunk16 = nchunk * 2

    @functools.partial(
        pl.kernel,
        mesh=mesh,
        out_type=jax.ShapeDtypeStruct((total, EMBED_DIM), jnp.float32),
        scratch_types=[
            pltpu.VMEM((nchunk * 2, CHUNK), jnp.int32),
            pltpu.VMEM((NBUF, CHUNK, EMBED_DIM), jnp.float32),
            pltpu.SemaphoreType.DMA((NBUF,)),
        ],
        compiler_params=pltpu.CompilerParams(use_tc_tiling_on_sc=False),
    )
    def gather_kernel(idx_hbm, table_hbm, out_hbm, idx_v, rows_v, gsem):
        @pl.when(lax.axis_index("c") == 0)
        def _all():
            wid = lax.axis_index("s")
            cbase = wid * nchunk16

            pltpu.sync_copy(idx_hbm.at[pl.ds(cbase, nchunk16)], idx_v)

            def start_gather(c, b):
                pltpu.async_copy(table_hbm.at[idx_v.at[c]], rows_v.at[b], gsem.at[b])

            for b in range(NBUF):
                start_gather(b, b)

            def body(jj, carry):
                for b in range(NBUF):
                    c = jj * NBUF + b
                    pltpu.make_async_copy(
                        table_hbm.at[pl.ds(0, CHUNK)], rows_v.at[b], gsem.at[b]
                    ).wait()
                    pltpu.sync_copy(
                        rows_v.at[b], out_hbm.at[pl.ds((cbase + c) * CHUNK, CHUNK)]
                    )
                    nxt = c + NBUF

                    @pl.when(nxt < nchunk16)
                    def _():
                        start_gather(nxt, b)

                return carry

            lax.fori_loop(0, nchunk16 // NBUF, body, 0)

    return gather_kernel


def kernel(token_ids, embeddings):
    b, h = token_ids.shape
    total = b * h
    flat_ids = token_ids.reshape(total // CHUNK, CHUNK).astype(jnp.int32)
    out = _build_gather(total)(flat_ids, embeddings)
    return out.reshape(b, h, EMBED_DIM)


# speedup vs baseline: 1.0093x; 1.0093x over previous
import functools

import jax
import jax.numpy as jnp
from jax import lax
from jax.experimental import pallas as pl
from jax.experimental.pallas import tpu as pltpu
from jax.experimental.pallas import tpu_sc as plsc

EMBED_DIM = 64
CHUNK = 128
NBUF = 5
BUFS = 2 * NBUF
NUM_CORES = 2
NUM_SUBCORES = 16
NUM_WORKERS = NUM_CORES * NUM_SUBCORES


@functools.lru_cache(maxsize=None)
def _build_gather(total: int):
    nchunk_total = total // CHUNK
    nchunk = nchunk_total // NUM_WORKERS
    assert nchunk % BUFS == 0
    mesh = plsc.VectorSubcoreMesh(core_axis_name="c", subcore_axis_name="s")

    @functools.partial(
        pl.kernel,
        mesh=mesh,
        out_type=jax.ShapeDtypeStruct((total, EMBED_DIM), jnp.float32),
        scratch_types=[
            pltpu.VMEM((nchunk, CHUNK), jnp.int32),
            pltpu.VMEM((BUFS, CHUNK, EMBED_DIM), jnp.float32),
            pltpu.SemaphoreType.DMA((BUFS,)),
            pltpu.SemaphoreType.DMA((BUFS,)),
        ],
        compiler_params=pltpu.CompilerParams(use_tc_tiling_on_sc=False),
    )
    def gather_kernel(idx_hbm, table_hbm, out_hbm, idx_v, rows_v, gsem, wsem):
        wid = lax.axis_index("c") * NUM_SUBCORES + lax.axis_index("s")
        cbase = wid * nchunk

        pltpu.sync_copy(idx_hbm.at[pl.ds(cbase, nchunk)], idx_v)

        def start_gather(c, b):
            pltpu.async_copy(table_hbm.at[idx_v.at[c]], rows_v.at[b], gsem.at[b])

        for b in range(NBUF):
            start_gather(b, b)

        def body(jj, carry):
            for i in range(BUFS):
                c = jj * BUFS + i
                pltpu.make_async_copy(
                    table_hbm.at[pl.ds(0, CHUNK)], rows_v.at[i], gsem.at[i]
                ).wait()
                pltpu.async_copy(
                    rows_v.at[i],
                    out_hbm.at[pl.ds((cbase + c) * CHUNK, CHUNK)],
                    wsem.at[i],
                )
                bn = (i + NBUF) % BUFS
                nxt = c + NBUF

                @pl.when(c >= NBUF)
                def _drain():
                    pltpu.make_async_copy(
                        rows_v.at[bn],
                        out_hbm.at[pl.ds(0, CHUNK)],
                        wsem.at[bn],
                    ).wait()

                @pl.when(nxt < nchunk)
                def _next():
                    start_gather(nxt, bn)

            return carry

        lax.fori_loop(0, nchunk // BUFS, body, 0)

        for k in range(NBUF):
            b = (nchunk - NBUF + k) % BUFS
            pltpu.make_async_copy(
                rows_v.at[b], out_hbm.at[pl.ds(0, CHUNK)], wsem.at[b]
            ).wait()

    return gather_kernel


def kernel(token_ids, embeddings):
    b, h = token_ids.shape
    total = b * h
    flat_ids = token_ids.reshape(total // CHUNK, CHUNK).astype(jnp.int32)
    out = _build_gather(total)(flat_ids, embeddings)
    return out.reshape(b, h, EMBED_DIM)
